# baseline (speedup 1.0000x reference)
import functools

import jax
import jax.numpy as jnp
from jax import lax
from jax.experimental import pallas as pl
from jax.experimental.pallas import tpu as pltpu
from jax.experimental.pallas import tpu_sc as plsc


TM = 128


def _sc_gather(x, order):
    N, D = x.shape
    info = plsc.get_sparse_core_info()
    NC, NS = info.num_cores, info.num_subcores
    NW = NC * NS
    rows_per_w = N // NW
    C = 16
    nch = rows_per_w // C
    idx3 = order.reshape(NW, nch, C)

    mesh = plsc.VectorSubcoreMesh(core_axis_name="c", subcore_axis_name="s")

    @functools.partial(
        pl.kernel, mesh=mesh,
        out_type=jax.ShapeDtypeStruct((N, D), x.dtype),
        compiler_params=pltpu.CompilerParams(use_tc_tiling_on_sc=True),
        scratch_types=[
            pltpu.VMEM((nch, C), jnp.int32),
            pltpu.VMEM((C, D), jnp.float32),
            pltpu.VMEM((C, D), jnp.float32),
            pltpu.SemaphoreType.DMA,
            pltpu.SemaphoreType.DMA,
            pltpu.SemaphoreType.DMA,
            pltpu.SemaphoreType.DMA,
        ],
    )
    def gk(x_hbm, idx_hbm, out_hbm, idx_v, buf0, buf1, g0, g1, s0, s1):
        wid = lax.axis_index("s") * NC + lax.axis_index("c")
        base = wid * rows_per_w
        pltpu.sync_copy(idx_hbm.at[wid], idx_v)
        bufs = (buf0, buf1)
        gsems = (g0, g1)
        ssems = (s0, s1)
        gathers = [None] * nch
        stores = [None] * nch
        for c in range(nch):
            b = c & 1
            if c >= 2:
                stores[c - 2].wait()
            gathers[c] = pltpu.async_copy(
                x_hbm.at[idx_v.at[c]], bufs[b], gsems[b])
            if c >= 1:
                gathers[c - 1].wait()
                stores[c - 1] = pltpu.async_copy(
                    bufs[1 - b],
                    out_hbm.at[pl.ds(base + (c - 1) * C, C)],
                    ssems[1 - b])
        last = nch - 1
        gathers[last].wait()
        stores[last] = pltpu.async_copy(
            bufs[last & 1],
            out_hbm.at[pl.ds(base + last * C, C)],
            ssems[last & 1])
        if nch >= 2:
            stores[last - 1].wait()
        stores[last].wait()

    return gk(x, idx3)


def _grouped_matmul(x_sorted, W, b3, work_tile, work_expert, work_start,
                    work_end, work_first, work_wchg, work_tchg,
                    *, T, WMAX, D):
    N = x_sorted.shape[0]
    OUT = W.shape[1]

    def body(tile_s, ex_s, st_s, en_s, fi_s, wc_s, tc_s,
             x_ref, w_ref, b_ref, o_ref, w16_ref, x16_ref):
        w = pl.program_id(0)

        @pl.when(wc_s[w] == 1)
        def _cvt_w():
            w16_ref[...] = w_ref[...].astype(jnp.bfloat16)

        @pl.when(tc_s[w] == 1)
        def _cvt_x():
            x16_ref[...] = x_ref[...].astype(jnp.bfloat16)

        y = lax.dot_general(
            x16_ref[...], w16_ref[...],
            dimension_numbers=(((1,), (1,)), ((), ())),
            preferred_element_type=jnp.float32,
        )
        y = y + b_ref[0, 0, :][None, :]
        rows = lax.broadcasted_iota(jnp.int32, (TM, OUT), 0)
        mask = (rows >= st_s[w]) & (rows < en_s[w])
        y = jnp.where(mask, y, 0.0)

        y3 = y[:, None, :]

        @pl.when(fi_s[w] == 1)
        def _init():
            o_ref[...] = y3

        @pl.when(fi_s[w] == 0)
        def _acc():
            o_ref[...] = o_ref[...] + y3

    grid_spec = pltpu.PrefetchScalarGridSpec(
        num_scalar_prefetch=7,
        grid=(WMAX,),
        in_specs=[
            pl.BlockSpec((TM, D),
                         lambda w, tile, ex, st, en, fi, wc, tc: (tile[w], 0)),
            pl.BlockSpec((OUT, D),
                         lambda w, tile, ex, st, en, fi, wc, tc: (ex[w], 0)),
            pl.BlockSpec((1, 1, OUT),
                         lambda w, tile, ex, st, en, fi, wc, tc: (ex[w], 0, 0)),
        ],
        out_specs=pl.BlockSpec(
            (TM, 1, OUT),
            lambda w, tile, ex, st, en, fi, wc, tc: (tile[w], 0, 0)),
        scratch_shapes=[
            pltpu.VMEM((OUT, D), jnp.bfloat16),
            pltpu.VMEM((TM, D), jnp.bfloat16),
        ],
    )
    return pl.pallas_call(
        body,
        grid_spec=grid_spec,
        out_shape=jax.ShapeDtypeStruct((N, 1, OUT), jnp.float32),
        compiler_params=pltpu.CompilerParams(
            dimension_semantics=("arbitrary",),
        ),
    )(work_tile, work_expert, work_start, work_end, work_first,
      work_wchg, work_tchg, x_sorted, W.reshape(-1, W.shape[2]), b3)


def kernel(x, subj_idx, W, b):
    N, D = x.shape
    E, OUT, _ = W.shape
    T = N // TM
    WMAX = T + E - 1

    subj_idx = subj_idx.astype(jnp.int32)
    order = jnp.argsort(subj_idx, stable=True)
    counts = jnp.bincount(subj_idx, length=E)
    ends = jnp.cumsum(counts)
    starts = ends - counts

    tile_lo = (jnp.arange(T, dtype=jnp.int32) * TM)[:, None]
    tile_hi = tile_lo + TM
    ov_lo = jnp.maximum(starts[None, :].astype(jnp.int32), tile_lo)
    ov_hi = jnp.minimum(ends[None, :].astype(jnp.int32), tile_hi)
    valid = ov_lo < ov_hi
    flat_valid = valid.reshape(-1)
    pos = jnp.cumsum(flat_valid) - 1
    slot = jnp.where(flat_valid, pos, WMAX)

    def fill(vals, default):
        out = jnp.full((WMAX,), default, jnp.int32)
        return out.at[slot].set(vals.reshape(-1).astype(jnp.int32),
                                mode='drop')

    t_ids = jnp.broadcast_to(jnp.arange(T, dtype=jnp.int32)[:, None], (T, E))
    e_ids = jnp.broadcast_to(jnp.arange(E, dtype=jnp.int32)[None, :], (T, E))
    work_tile = fill(t_ids, T - 1)
    work_expert = fill(e_ids, E - 1)
    work_start = fill(ov_lo - tile_lo, 0)
    work_end = fill(ov_hi - tile_lo, 0)
    work_first = fill(valid & (jnp.cumsum(valid, axis=1) == 1), 0)
    prev_ex = jnp.concatenate([jnp.array([-1], jnp.int32), work_expert[:-1]])
    prev_t = jnp.concatenate([jnp.array([-1], jnp.int32), work_tile[:-1]])
    work_wchg = (work_expert != prev_ex).astype(jnp.int32)
    work_tchg = (work_tile != prev_t).astype(jnp.int32)

    x_sorted = _sc_gather(x, order)
    return _grouped_matmul(x_sorted, W, b.reshape(E, 1, OUT),
                           work_tile, work_expert, work_start, work_end,
                           work_first, work_wchg, work_tchg,
                           T=T, WMAX=WMAX, D=D)


# speedup vs baseline: 1.4839x; 1.4839x over previous
import functools

import jax
import jax.numpy as jnp
from jax import lax
from jax.experimental import pallas as pl
from jax.experimental.pallas import tpu as pltpu
from jax.experimental.pallas import tpu_sc as plsc


TM = 256


def _sc_gather(x, order):
    N, D = x.shape
    info = plsc.get_sparse_core_info()
    NC, NS = info.num_cores, info.num_subcores
    NW = NC * NS
    rows_per_w = N // NW
    C = 16
    nch = rows_per_w // C
    idx3 = order.reshape(NW, nch, C)

    mesh = plsc.VectorSubcoreMesh(core_axis_name="c", subcore_axis_name="s")

    @functools.partial(
        pl.kernel, mesh=mesh,
        out_type=jax.ShapeDtypeStruct((N, D), x.dtype),
        compiler_params=pltpu.CompilerParams(use_tc_tiling_on_sc=True),
        scratch_types=[
            pltpu.VMEM((nch, C), jnp.int32),
            pltpu.VMEM((C, D), jnp.float32),
            pltpu.VMEM((C, D), jnp.float32),
            pltpu.SemaphoreType.DMA,
            pltpu.SemaphoreType.DMA,
            pltpu.SemaphoreType.DMA,
            pltpu.SemaphoreType.DMA,
        ],
    )
    def gk(x_hbm, idx_hbm, out_hbm, idx_v, buf0, buf1, g0, g1, s0, s1):
        wid = lax.axis_index("s") * NC + lax.axis_index("c")
        base = wid * rows_per_w
        pltpu.sync_copy(idx_hbm.at[wid], idx_v)
        bufs = (buf0, buf1)
        gsems = (g0, g1)
        ssems = (s0, s1)
        gathers = [None] * nch
        stores = [None] * nch
        for c in range(nch):
            b = c & 1
            if c >= 2:
                stores[c - 2].wait()
            gathers[c] = pltpu.async_copy(
                x_hbm.at[idx_v.at[c]], bufs[b], gsems[b])
            if c >= 1:
                gathers[c - 1].wait()
                stores[c - 1] = pltpu.async_copy(
                    bufs[1 - b],
                    out_hbm.at[pl.ds(base + (c - 1) * C, C)],
                    ssems[1 - b])
        last = nch - 1
        gathers[last].wait()
        stores[last] = pltpu.async_copy(
            bufs[last & 1],
            out_hbm.at[pl.ds(base + last * C, C)],
            ssems[last & 1])
        if nch >= 2:
            stores[last - 1].wait()
        stores[last].wait()

    return gk(x, idx3)


def _grouped_matmul(x_sorted, W, b3, work_tile, work_expert, work_start,
                    work_end, work_first, work_wchg, work_tchg,
                    *, T, WMAX, D):
    N = x_sorted.shape[0]
    OUT = W.shape[1]

    def body(tile_s, ex_s, st_s, en_s, fi_s, wc_s, tc_s,
             x_ref, w_ref, b_ref, o_ref):
        w = pl.program_id(0)
        y = lax.dot_general(
            x_ref[...].astype(jnp.bfloat16), w_ref[...].astype(jnp.bfloat16),
            dimension_numbers=(((1,), (1,)), ((), ())),
            preferred_element_type=jnp.float32,
        )
        y = y + b_ref[0, 0, :][None, :]
        rows = lax.broadcasted_iota(jnp.int32, (TM, OUT), 0)
        mask = (rows >= st_s[w]) & (rows < en_s[w])
        y = jnp.where(mask, y, 0.0)

        y3 = y[:, None, :]

        @pl.when(fi_s[w] == 1)
        def _init():
            o_ref[...] = y3

        @pl.when(fi_s[w] == 0)
        def _acc():
            o_ref[...] = o_ref[...] + y3

    grid_spec = pltpu.PrefetchScalarGridSpec(
        num_scalar_prefetch=7,
        grid=(WMAX,),
        in_specs=[
            pl.BlockSpec((TM, D),
                         lambda w, tile, ex, st, en, fi, wc, tc: (tile[w], 0)),
            pl.BlockSpec((OUT, D),
                         lambda w, tile, ex, st, en, fi, wc, tc: (ex[w], 0)),
            pl.BlockSpec((1, 1, OUT),
                         lambda w, tile, ex, st, en, fi, wc, tc: (ex[w], 0, 0)),
        ],
        out_specs=pl.BlockSpec(
            (TM, 1, OUT),
            lambda w, tile, ex, st, en, fi, wc, tc: (tile[w], 0, 0)),
    )
    return pl.pallas_call(
        body,
        grid_spec=grid_spec,
        out_shape=jax.ShapeDtypeStruct((N, 1, OUT), jnp.float32),
        compiler_params=pltpu.CompilerParams(
            dimension_semantics=("arbitrary",),
        ),
    )(work_tile, work_expert, work_start, work_end, work_first,
      work_wchg, work_tchg, x_sorted, W.reshape(-1, W.shape[2]), b3)


def kernel(x, subj_idx, W, b):
    N, D = x.shape
    E, OUT, _ = W.shape
    T = N // TM
    WMAX = T + E - 1

    subj_idx = subj_idx.astype(jnp.int32)
    order = jnp.argsort(subj_idx, stable=True)
    counts = jnp.bincount(subj_idx, length=E)
    ends = jnp.cumsum(counts)
    starts = ends - counts

    tile_lo = (jnp.arange(T, dtype=jnp.int32) * TM)[:, None]
    tile_hi = tile_lo + TM
    ov_lo = jnp.maximum(starts[None, :].astype(jnp.int32), tile_lo)
    ov_hi = jnp.minimum(ends[None, :].astype(jnp.int32), tile_hi)
    valid = ov_lo < ov_hi
    flat_valid = valid.reshape(-1)
    pos = jnp.cumsum(flat_valid) - 1
    slot = jnp.where(flat_valid, pos, WMAX)

    def fill(vals, default):
        out = jnp.full((WMAX,), default, jnp.int32)
        return out.at[slot].set(vals.reshape(-1).astype(jnp.int32),
                                mode='drop')

    t_ids = jnp.broadcast_to(jnp.arange(T, dtype=jnp.int32)[:, None], (T, E))
    e_ids = jnp.broadcast_to(jnp.arange(E, dtype=jnp.int32)[None, :], (T, E))
    work_tile = fill(t_ids, T - 1)
    work_expert = fill(e_ids, E - 1)
    work_start = fill(ov_lo - tile_lo, 0)
    work_end = fill(ov_hi - tile_lo, 0)
    work_first = fill(valid & (jnp.cumsum(valid, axis=1) == 1), 0)
    prev_ex = jnp.concatenate([jnp.array([-1], jnp.int32), work_expert[:-1]])
    prev_t = jnp.concatenate([jnp.array([-1], jnp.int32), work_tile[:-1]])
    work_wchg = (work_expert != prev_ex).astype(jnp.int32)
    work_tchg = (work_tile != prev_t).astype(jnp.int32)

    x_sorted = _sc_gather(x, order)
    return _grouped_matmul(x_sorted, W, b.reshape(E, 1, OUT),
                           work_tile, work_expert, work_start, work_end,
                           work_first, work_wchg, work_tchg,
                           T=T, WMAX=WMAX, D=D)


# speedup vs baseline: 1.4865x; 1.0017x over previous
import functools

import jax
import jax.numpy as jnp
from jax import lax
from jax.experimental import pallas as pl
from jax.experimental.pallas import tpu as pltpu
from jax.experimental.pallas import tpu_sc as plsc


TM = 256


def _sc_gather(x, order):
    N, D = x.shape
    info = plsc.get_sparse_core_info()
    NC, NS = info.num_cores, info.num_subcores
    NW = NC * NS
    rows_per_w = N // NW
    C = 16
    nch = rows_per_w // C
    idx3 = order.reshape(NW, nch, C)

    mesh = plsc.VectorSubcoreMesh(core_axis_name="c", subcore_axis_name="s")

    @functools.partial(
        pl.kernel, mesh=mesh,
        out_type=jax.ShapeDtypeStruct((N, D), x.dtype),
        compiler_params=pltpu.CompilerParams(use_tc_tiling_on_sc=True),
        scratch_types=[
            pltpu.VMEM((nch, C), jnp.int32),
            pltpu.VMEM((C, D), jnp.float32),
            pltpu.VMEM((C, D), jnp.float32),
            pltpu.SemaphoreType.DMA,
            pltpu.SemaphoreType.DMA,
            pltpu.SemaphoreType.DMA,
            pltpu.SemaphoreType.DMA,
        ],
    )
    def gk(x_hbm, idx_hbm, out_hbm, idx_v, buf0, buf1, g0, g1, s0, s1):
        wid = lax.axis_index("s") * NC + lax.axis_index("c")
        base = wid * rows_per_w
        pltpu.sync_copy(idx_hbm.at[wid], idx_v)
        bufs = (buf0, buf1)
        gsems = (g0, g1)
        ssems = (s0, s1)
        gathers = [None] * nch
        stores = [None] * nch
        for c in range(nch):
            b = c & 1
            if c >= 2:
                stores[c - 2].wait()
            gathers[c] = pltpu.async_copy(
                x_hbm.at[idx_v.at[c]], bufs[b], gsems[b])
            if c >= 1:
                gathers[c - 1].wait()
                stores[c - 1] = pltpu.async_copy(
                    bufs[1 - b],
                    out_hbm.at[pl.ds(base + (c - 1) * C, C)],
                    ssems[1 - b])
        last = nch - 1
        gathers[last].wait()
        stores[last] = pltpu.async_copy(
            bufs[last & 1],
            out_hbm.at[pl.ds(base + last * C, C)],
            ssems[last & 1])
        if nch >= 2:
            stores[last - 1].wait()
        stores[last].wait()

    return gk(x, idx3)


def _grouped_matmul(x_sorted, W, b3, work_tile, work_expert, work_start,
                    work_end, work_first, *, T, WMAX, D):
    N = x_sorted.shape[0]
    OUT = W.shape[1]

    def body(tile_s, ex_s, st_s, en_s, fi_s, x_ref, w_ref, b_ref, o_ref):
        w = pl.program_id(0)
        y = lax.dot_general(
            x_ref[...].astype(jnp.bfloat16), w_ref[...].astype(jnp.bfloat16),
            dimension_numbers=(((1,), (1,)), ((), ())),
            preferred_element_type=jnp.float32,
        )
        y = y + b_ref[0, 0, :][None, :]
        rows = lax.broadcasted_iota(jnp.int32, (TM, OUT), 0)
        mask = (rows >= st_s[w]) & (rows < en_s[w])
        y = jnp.where(mask, y, 0.0)

        y3 = y[:, None, :]

        @pl.when(fi_s[w] == 1)
        def _init():
            o_ref[...] = y3

        @pl.when(fi_s[w] == 0)
        def _acc():
            o_ref[...] = o_ref[...] + y3

    grid_spec = pltpu.PrefetchScalarGridSpec(
        num_scalar_prefetch=5,
        grid=(WMAX,),
        in_specs=[
            pl.BlockSpec((TM, D),
                         lambda w, tile, ex, st, en, fi: (tile[w], 0)),
            pl.BlockSpec((OUT, D),
                         lambda w, tile, ex, st, en, fi: (ex[w], 0)),
            pl.BlockSpec((1, 1, OUT),
                         lambda w, tile, ex, st, en, fi: (ex[w], 0, 0)),
        ],
        out_specs=pl.BlockSpec(
            (TM, 1, OUT),
            lambda w, tile, ex, st, en, fi: (tile[w], 0, 0)),
    )
    return pl.pallas_call(
        body,
        grid_spec=grid_spec,
        out_shape=jax.ShapeDtypeStruct((N, 1, OUT), jnp.float32),
        compiler_params=pltpu.CompilerParams(
            dimension_semantics=("arbitrary",),
        ),
    )(work_tile, work_expert, work_start, work_end, work_first,
      x_sorted, W.reshape(-1, W.shape[2]), b3)


def kernel(x, subj_idx, W, b):
    N, D = x.shape
    E, OUT, _ = W.shape
    T = N // TM
    WMAX = T + E - 1

    subj_idx = subj_idx.astype(jnp.int32)
    order = jnp.argsort(subj_idx, stable=True)
    counts = jnp.bincount(subj_idx, length=E)
    ends = jnp.cumsum(counts)
    starts = ends - counts

    tile_lo = (jnp.arange(T, dtype=jnp.int32) * TM)[:, None]
    tile_hi = tile_lo + TM
    ov_lo = jnp.maximum(starts[None, :].astype(jnp.int32), tile_lo)
    ov_hi = jnp.minimum(ends[None, :].astype(jnp.int32), tile_hi)
    valid = ov_lo < ov_hi
    flat_valid = valid.reshape(-1)
    pos = jnp.cumsum(flat_valid) - 1
    slot = jnp.where(flat_valid, pos, WMAX)

    def fill(vals, default):
        out = jnp.full((WMAX,), default, jnp.int32)
        return out.at[slot].set(vals.reshape(-1).astype(jnp.int32),
                                mode='drop')

    t_ids = jnp.broadcast_to(jnp.arange(T, dtype=jnp.int32)[:, None], (T, E))
    e_ids = jnp.broadcast_to(jnp.arange(E, dtype=jnp.int32)[None, :], (T, E))
    work_tile = fill(t_ids, T - 1)
    work_expert = fill(e_ids, E - 1)
    work_start = fill(ov_lo - tile_lo, 0)
    work_end = fill(ov_hi - tile_lo, 0)
    work_first = fill(valid & (jnp.cumsum(valid, axis=1) == 1), 0)

    x_sorted = _sc_gather(x, order)
    return _grouped_matmul(x_sorted, W, b.reshape(E, 1, OUT),
                           work_tile, work_expert, work_start, work_end,
                           work_first, T=T, WMAX=WMAX, D=D)
